# baseline (speedup 1.0000x reference)
import jax
import jax.numpy as jnp
from jax.experimental import pallas as pl

_TEMPERATURE = 1.0
_BLOCK_ROWS = 1024


def _scale_kernel(x_ref, o_ref):
    o_ref[...] = x_ref[...] * jnp.float32(1.0 / _TEMPERATURE)


def kernel(input, labels):
    rows, cols = input.shape
    return pl.pallas_call(
        _scale_kernel,
        grid=(rows // _BLOCK_ROWS,),
        in_specs=[pl.BlockSpec((_BLOCK_ROWS, cols), lambda i: (i, 0))],
        out_specs=pl.BlockSpec((_BLOCK_ROWS, cols), lambda i: (i, 0)),
        out_shape=jax.ShapeDtypeStruct((rows, cols), input.dtype),
    )(input)
